# baseline (speedup 1.0000x reference)
import jax
import jax.numpy as jnp
from jax import lax
from jax.experimental import pallas as pl
from jax.experimental.pallas import tpu as pltpu
from jax.experimental.pallas import tpu_sc as plsc

N = 10000
NP = 10240
E0 = 100000
BATCH = 128
NBATCH = 50
NBG = NBATCH + 2
EPT = BATCH * NBATCH
EP = 16 * EPT
EPT_B = EP // 32
F = 448
FC = 112
STRIPE = NP // 16
NB = 20
NBS = 512
NB2 = 25
NBS2 = 400


def _deg_body(dst_hbm, zeros1_hbm, deg_out, dstv, accv):
    c = lax.axis_index("c")
    s = lax.axis_index("s")
    w = c * 16 + s
    pltpu.sync_copy(zeros1_hbm, accv)
    pltpu.sync_copy(dst_hbm.at[pl.ds(w * EPT_B, EPT_B)], dstv)

    def body(k, carry):
        idx = dstv[pl.ds(k * 16, 16)]
        plsc.addupdate_scatter(accv, [idx], jnp.full((16,), 1.0, jnp.float32))
        return carry

    lax.fori_loop(0, EPT_B // 16, body, 0)
    pltpu.sync_copy(accv, deg_out.at[w])


def _make_deg_kernel(mesh):
    return pl.kernel(
        _deg_body,
        out_type=jax.ShapeDtypeStruct((32, NP), jnp.float32),
        mesh=mesh,
        scratch_types=[
            pltpu.VMEM((EPT_B,), jnp.int32),
            pltpu.VMEM((NP,), jnp.float32),
        ],
        compiler_params=pltpu.CompilerParams(needs_layout_passes=False),
    )


def _mp_body(src_hbm, dst_hbm, mp0, mp1, mp2, mp3, dinv_hbm, zrows_hbm,
             zeros1_hbm, h0_out, h1_out, h2_out, h3_out, cacc_out,
             srcv, dstv, rows0, rows1, valsv, sem0, sem1, acc_sh, cacc_sh):
    c = lax.axis_index("c")
    s = lax.axis_index("s")
    stripe = pl.ds(s * STRIPE, STRIPE)
    pltpu.sync_copy(src_hbm.at[s], srcv)
    pltpu.sync_copy(dst_hbm.at[s], dstv)

    def run_chunk(mp_hbm, h_out):
        pltpu.sync_copy(zrows_hbm, acc_sh.at[stripe])
        plsc.subcore_barrier()
        pltpu.async_copy(mp_hbm.at[srcv.at[0]], rows0, sem0)
        pltpu.async_copy(mp_hbm.at[srcv.at[1]], rows1, sem1)

        def body(k, carry):
            j = 2 * k
            pltpu.make_async_copy(mp_hbm.at[srcv.at[j]], rows0, sem0).wait()
            pltpu.sync_copy(rows0, acc_sh.at[dstv.at[j]], add=True)
            pltpu.async_copy(mp_hbm.at[srcv.at[j + 2]], rows0, sem0)
            pltpu.make_async_copy(mp_hbm.at[srcv.at[j + 1]], rows1,
                                  sem1).wait()
            pltpu.sync_copy(rows1, acc_sh.at[dstv.at[j + 1]], add=True)
            pltpu.async_copy(mp_hbm.at[srcv.at[j + 3]], rows1, sem1)
            return carry

        lax.fori_loop(0, NBATCH // 2, body, 0)
        pltpu.make_async_copy(mp_hbm.at[srcv.at[NBATCH]], rows0, sem0).wait()
        pltpu.make_async_copy(mp_hbm.at[srcv.at[NBATCH + 1]], rows1,
                              sem1).wait()
        plsc.subcore_barrier()
        pltpu.sync_copy(acc_sh.at[stripe], h_out.at[stripe])
        plsc.subcore_barrier()

    def run_cacc(j_lo, row):
        pltpu.sync_copy(zeros1_hbm.at[stripe], cacc_sh.at[stripe])
        plsc.subcore_barrier()

        def body(j, carry):
            pltpu.async_copy(dinv_hbm.at[dstv.at[j]], valsv, sem0).wait()
            pltpu.sync_copy(valsv, cacc_sh.at[srcv.at[j]], add=True)
            return carry

        lax.fori_loop(j_lo, j_lo + NBATCH // 2, body, 0)
        plsc.subcore_barrier()
        pltpu.sync_copy(cacc_sh.at[stripe], cacc_out.at[row, stripe])

    @pl.when(c == 0)
    def _():
        run_chunk(mp0, h0_out)
        run_chunk(mp1, h1_out)
        run_cacc(0, 0)

    @pl.when(c == 1)
    def _():
        run_chunk(mp2, h2_out)
        run_chunk(mp3, h3_out)
        run_cacc(NBATCH // 2, 1)


def _make_mp_kernel(mesh):
    return pl.kernel(
        _mp_body,
        out_type=(
            jax.ShapeDtypeStruct((NP, FC), jnp.float32),
            jax.ShapeDtypeStruct((NP, FC), jnp.float32),
            jax.ShapeDtypeStruct((NP, FC), jnp.float32),
            jax.ShapeDtypeStruct((NP, FC), jnp.float32),
            jax.ShapeDtypeStruct((2, NP), jnp.float32),
        ),
        mesh=mesh,
        scratch_types=[
            pltpu.VMEM((NBG, BATCH), jnp.int32),
            pltpu.VMEM((NBG, BATCH), jnp.int32),
            pltpu.VMEM((BATCH, FC), jnp.float32),
            pltpu.VMEM((BATCH, FC), jnp.float32),
            pltpu.VMEM((BATCH,), jnp.float32),
            pltpu.SemaphoreType.DMA,
            pltpu.SemaphoreType.DMA,
            pltpu.VMEM_SHARED((NP, FC), jnp.float32),
            pltpu.VMEM_SHARED((NP,), jnp.float32),
        ],
        compiler_params=pltpu.CompilerParams(needs_layout_passes=False,
                                             use_tc_tiling_on_sc=False),
    )


def _dinv_body(dp_ref, dv_ref):
    deg = jnp.sum(dp_ref[...], axis=0, keepdims=True) + 1.0
    dv_ref[...] = lax.rsqrt(deg)


def _mmscale_body(x_ref, w1d_ref, dinv_ref, mp0_ref, mp1_ref, mp2_ref,
                  mp3_ref):
    mm = dinv_ref[...] * jnp.dot(x_ref[...], w1d_ref[...],
                                 preferred_element_type=jnp.float32)
    mp0_ref[...] = mm[:, 0 * FC:1 * FC]
    mp1_ref[...] = mm[:, 1 * FC:2 * FC]
    mp2_ref[...] = mm[:, 2 * FC:3 * FC]
    mp3_ref[...] = mm[:, 3 * FC:4 * FC]


def _w_body(cacc_ref, dinv_ref, out_ref):
    dv = dinv_ref[...]
    w = dv * jnp.sum(cacc_ref[...], axis=0, keepdims=True) + dv * dv
    lane = lax.broadcasted_iota(jnp.int32, (1, NP), 1)
    w = jnp.where(lane < N, w, 0.0)
    out_ref[...] = jnp.concatenate([dv, w], axis=0)


def _red_body(h1_ref, mp_ref, scal_ref, b1_ref, out_ref):
    i = pl.program_id(0)
    dinv = scal_ref[0, pl.ds(i * NBS, NBS)]
    w = scal_ref[1, pl.ds(i * NBS, NBS)]
    h1 = dinv[:, None] * (h1_ref[...] + mp_ref[...]) + b1_ref[...]
    rl = jnp.maximum(h1, 0.0)
    rows = i * NBS + lax.broadcasted_iota(jnp.int32, (NBS, FC), 0)
    rl = jnp.where(rows < N, rl, 0.0)

    @pl.when(i == 0)
    def _():
        out_ref[...] = jnp.zeros_like(out_ref)

    out_ref[...] += jnp.dot(w[None, :], rl, preferred_element_type=jnp.float32)


def _fin_body(r_ref, w2_ref, b2_ref, wfc_ref, bfc_ref, o_ref):
    h = jnp.dot(r_ref[...], w2_ref[...],
                preferred_element_type=jnp.float32) / N + b2_ref[...]
    val = jnp.sum(h * wfc_ref[...]) + bfc_ref[0, 0]
    o_ref[...] = jax.nn.sigmoid(val.reshape(1, 1) / 28.0)


def kernel(x, edge_index, W1, b1, W2, b2, Wfc, bfc):
    src = edge_index[0].astype(jnp.int32)
    dst = edge_index[1].astype(jnp.int32)
    pad = jnp.full((EP - E0,), N, jnp.int32)
    srcp = jnp.concatenate([src, pad])
    dstp = jnp.concatenate([dst, pad])
    dummy = jnp.full((16, NBG - NBATCH, BATCH), N, jnp.int32)
    src3d = jnp.concatenate([srcp.reshape(16, NBATCH, BATCH), dummy], axis=1)
    dst3d = jnp.concatenate([dstp.reshape(16, NBATCH, BATCH), dummy], axis=1)
    zeros1 = jnp.zeros((NP,), jnp.float32)
    zrows = jnp.zeros((STRIPE, FC), jnp.float32)

    mesh = plsc.VectorSubcoreMesh(core_axis_name="c", subcore_axis_name="s",
                                  num_cores=2, num_subcores=16)
    degparts = _make_deg_kernel(mesh)(dstp, zeros1)

    dinv = pl.pallas_call(
        _dinv_body,
        in_specs=[pl.BlockSpec((32, NP), lambda: (0, 0))],
        out_specs=pl.BlockSpec((1, NP), lambda: (0, 0)),
        out_shape=jax.ShapeDtypeStruct((1, NP), jnp.float32),
    )(degparts)
    dinv1d = dinv.reshape(NP)

    w1d = jnp.kron(jnp.eye(28, dtype=jnp.float32), W1.astype(jnp.float32))
    mp_spec = pl.BlockSpec((NBS2, FC), lambda i: (i, 0))
    mps = pl.pallas_call(
        _mmscale_body,
        grid=(NB2,),
        in_specs=[pl.BlockSpec((NBS2, 784), lambda i: (i, 0)),
                  pl.BlockSpec((784, F), lambda i: (0, 0)),
                  pl.BlockSpec((NBS2, 1), lambda i: (i, 0))],
        out_specs=[mp_spec, mp_spec, mp_spec, mp_spec],
        out_shape=[jax.ShapeDtypeStruct((NP, FC), jnp.float32)] * 4,
    )(x, w1d, dinv.reshape(NP, 1))

    h0, h1, h2, h3, caccparts = _make_mp_kernel(mesh)(
        src3d, dst3d, mps[0], mps[1], mps[2], mps[3], dinv1d, zrows, zeros1)

    scal = pl.pallas_call(
        _w_body,
        in_specs=[pl.BlockSpec((2, NP), lambda: (0, 0)),
                  pl.BlockSpec((1, NP), lambda: (0, 0))],
        out_specs=pl.BlockSpec((2, NP), lambda: (0, 0)),
        out_shape=jax.ShapeDtypeStruct((2, NP), jnp.float32),
    )(caccparts, dinv)

    b1tile = jnp.tile(b1.astype(jnp.float32), 28)
    rs = []
    for cch, h_c in enumerate((h0, h1, h2, h3)):
        r_c = pl.pallas_call(
            _red_body,
            grid=(NB,),
            in_specs=[pl.BlockSpec((NBS, FC), lambda i: (i, 0)),
                      pl.BlockSpec((NBS, FC), lambda i: (i, 0)),
                      pl.BlockSpec((2, NP), lambda i: (0, 0)),
                      pl.BlockSpec((1, FC), lambda i: (0, 0))],
            out_specs=pl.BlockSpec((1, FC), lambda i: (0, 0)),
            out_shape=jax.ShapeDtypeStruct((1, FC), jnp.float32),
        )(h_c, mps[cch], scal,
          b1tile[cch * FC:(cch + 1) * FC].reshape(1, FC))
        rs.append(r_c)

    r28 = jnp.concatenate(rs, axis=1).reshape(28, 16)
    out = pl.pallas_call(
        _fin_body,
        in_specs=[pl.BlockSpec((28, 16), lambda: (0, 0)),
                  pl.BlockSpec((16, 32), lambda: (0, 0)),
                  pl.BlockSpec((1, 32), lambda: (0, 0)),
                  pl.BlockSpec((28, 32), lambda: (0, 0)),
                  pl.BlockSpec((1, 1), lambda: (0, 0))],
        out_specs=pl.BlockSpec((1, 1), lambda: (0, 0)),
        out_shape=jax.ShapeDtypeStruct((1, 1), jnp.float32),
    )(r28, W2.astype(jnp.float32), b2.reshape(1, 32),
      Wfc.reshape(28, 32), bfc.reshape(1, 1))
    return out


# speedup vs baseline: 1.3422x; 1.3422x over previous
import jax
import jax.numpy as jnp
from jax import lax
from jax.experimental import pallas as pl
from jax.experimental.pallas import tpu as pltpu
from jax.experimental.pallas import tpu_sc as plsc

N = 10000
NP = 10240
E0 = 100000
BATCH = 128
NBATCH = 50
NBG = NBATCH + 2
EPT = BATCH * NBATCH
EP = 16 * EPT
EPT_B = EP // 32
F = 448
FC = 112
STRIPE = NP // 16
NB = 20
NBS = 512
NB2 = 25
NBS2 = 400


def _deg_body(dst_hbm, zeros1_hbm, deg_out, dstv, accv):
    c = lax.axis_index("c")
    s = lax.axis_index("s")
    w = c * 16 + s
    pltpu.sync_copy(zeros1_hbm, accv)
    pltpu.sync_copy(dst_hbm.at[pl.ds(w * EPT_B, EPT_B)], dstv)

    def body(k, carry):
        idx = dstv[pl.ds(k * 16, 16)]
        plsc.addupdate_scatter(accv, [idx], jnp.full((16,), 1.0, jnp.float32))
        return carry

    lax.fori_loop(0, EPT_B // 16, body, 0)
    pltpu.sync_copy(accv, deg_out.at[w])


def _make_deg_kernel(mesh):
    return pl.kernel(
        _deg_body,
        out_type=jax.ShapeDtypeStruct((32, NP), jnp.float32),
        mesh=mesh,
        scratch_types=[
            pltpu.VMEM((EPT_B,), jnp.int32),
            pltpu.VMEM((NP,), jnp.float32),
        ],
        compiler_params=pltpu.CompilerParams(needs_layout_passes=False),
    )


def _mp_body(src_hbm, dst_hbm, mp0, mp1, mp2, mp3, dinv_hbm, zrows_hbm,
             zeros1_hbm, h0_out, h1_out, h2_out, h3_out, cacc_out,
             srcv, dstv, rows0, rows1, valsv, sem0, sem1, acc_sh, cacc_sh):
    c = lax.axis_index("c")
    s = lax.axis_index("s")
    stripe = pl.ds(s * STRIPE, STRIPE)
    pltpu.sync_copy(src_hbm.at[s], srcv)
    pltpu.sync_copy(dst_hbm.at[s], dstv)

    def run_chunk(mp_hbm, h_out):
        pltpu.sync_copy(zrows_hbm, acc_sh.at[stripe])
        plsc.subcore_barrier()
        def body(k, carry):
            j = 2 * k
            pltpu.async_copy(mp_hbm.at[srcv.at[j]], rows0, sem0)
            pltpu.async_copy(mp_hbm.at[srcv.at[j + 1]], rows1, sem0)
            pltpu.make_async_copy(mp_hbm.at[srcv.at[j]], rows0, sem0).wait()
            pltpu.make_async_copy(mp_hbm.at[srcv.at[j + 1]], rows1,
                                  sem0).wait()
            pltpu.sync_copy(rows0, acc_sh.at[dstv.at[j]], add=True)
            pltpu.sync_copy(rows1, acc_sh.at[dstv.at[j + 1]], add=True)
            return carry

        lax.fori_loop(0, NBATCH // 2, body, 0)
        plsc.subcore_barrier()
        pltpu.sync_copy(acc_sh.at[stripe], h_out.at[stripe])
        plsc.subcore_barrier()

    def run_cacc(j_lo, row):
        pltpu.sync_copy(zeros1_hbm.at[stripe], cacc_sh.at[stripe])
        plsc.subcore_barrier()

        def body(j, carry):
            pltpu.async_copy(dinv_hbm.at[dstv.at[j]], valsv, sem0).wait()
            pltpu.sync_copy(valsv, cacc_sh.at[srcv.at[j]], add=True)
            return carry

        lax.fori_loop(j_lo, j_lo + NBATCH // 2, body, 0)
        plsc.subcore_barrier()
        pltpu.sync_copy(cacc_sh.at[stripe], cacc_out.at[row, stripe])

    @pl.when(c == 0)
    def _():
        run_chunk(mp0, h0_out)
        run_chunk(mp1, h1_out)
        run_cacc(0, 0)

    @pl.when(c == 1)
    def _():
        run_chunk(mp2, h2_out)
        run_chunk(mp3, h3_out)
        run_cacc(NBATCH // 2, 1)


def _make_mp_kernel(mesh):
    return pl.kernel(
        _mp_body,
        out_type=(
            jax.ShapeDtypeStruct((NP, FC), jnp.float32),
            jax.ShapeDtypeStruct((NP, FC), jnp.float32),
            jax.ShapeDtypeStruct((NP, FC), jnp.float32),
            jax.ShapeDtypeStruct((NP, FC), jnp.float32),
            jax.ShapeDtypeStruct((2, NP), jnp.float32),
        ),
        mesh=mesh,
        scratch_types=[
            pltpu.VMEM((NBG, BATCH), jnp.int32),
            pltpu.VMEM((NBG, BATCH), jnp.int32),
            pltpu.VMEM((BATCH, FC), jnp.float32),
            pltpu.VMEM((BATCH, FC), jnp.float32),
            pltpu.VMEM((BATCH,), jnp.float32),
            pltpu.SemaphoreType.DMA,
            pltpu.SemaphoreType.DMA,
            pltpu.VMEM_SHARED((NP, FC), jnp.float32),
            pltpu.VMEM_SHARED((NP,), jnp.float32),
        ],
        compiler_params=pltpu.CompilerParams(needs_layout_passes=False,
                                             use_tc_tiling_on_sc=False),
    )


def _dinv_body(dp_ref, dv_ref):
    deg = jnp.sum(dp_ref[...], axis=0, keepdims=True) + 1.0
    dv_ref[...] = lax.rsqrt(deg)


def _mmscale_body(x_ref, w1d_ref, dinv_ref, mp0_ref, mp1_ref, mp2_ref,
                  mp3_ref):
    mm = dinv_ref[...] * jnp.dot(x_ref[...], w1d_ref[...],
                                 preferred_element_type=jnp.float32)
    mp0_ref[...] = mm[:, 0 * FC:1 * FC]
    mp1_ref[...] = mm[:, 1 * FC:2 * FC]
    mp2_ref[...] = mm[:, 2 * FC:3 * FC]
    mp3_ref[...] = mm[:, 3 * FC:4 * FC]


def _w_body(cacc_ref, dinv_ref, out_ref):
    dv = dinv_ref[...]
    w = dv * jnp.sum(cacc_ref[...], axis=0, keepdims=True) + dv * dv
    lane = lax.broadcasted_iota(jnp.int32, (1, NP), 1)
    w = jnp.where(lane < N, w, 0.0)
    out_ref[...] = jnp.concatenate([dv, w], axis=0)


def _red_body(h1_ref, mp_ref, scal_ref, b1_ref, out_ref):
    i = pl.program_id(0)
    dinv = scal_ref[0, pl.ds(i * NBS, NBS)]
    w = scal_ref[1, pl.ds(i * NBS, NBS)]
    h1 = dinv[:, None] * (h1_ref[...] + mp_ref[...]) + b1_ref[...]
    rl = jnp.maximum(h1, 0.0)
    rows = i * NBS + lax.broadcasted_iota(jnp.int32, (NBS, FC), 0)
    rl = jnp.where(rows < N, rl, 0.0)

    @pl.when(i == 0)
    def _():
        out_ref[...] = jnp.zeros_like(out_ref)

    out_ref[...] += jnp.dot(w[None, :], rl, preferred_element_type=jnp.float32)


def _fin_body(r_ref, w2_ref, b2_ref, wfc_ref, bfc_ref, o_ref):
    h = jnp.dot(r_ref[...], w2_ref[...],
                preferred_element_type=jnp.float32) / N + b2_ref[...]
    val = jnp.sum(h * wfc_ref[...]) + bfc_ref[0, 0]
    o_ref[...] = jax.nn.sigmoid(val.reshape(1, 1) / 28.0)


def kernel(x, edge_index, W1, b1, W2, b2, Wfc, bfc):
    src = edge_index[0].astype(jnp.int32)
    dst = edge_index[1].astype(jnp.int32)
    pad = jnp.full((EP - E0,), N, jnp.int32)
    srcp = jnp.concatenate([src, pad])
    dstp = jnp.concatenate([dst, pad])
    dummy = jnp.full((16, NBG - NBATCH, BATCH), N, jnp.int32)
    src3d = jnp.concatenate([srcp.reshape(16, NBATCH, BATCH), dummy], axis=1)
    dst3d = jnp.concatenate([dstp.reshape(16, NBATCH, BATCH), dummy], axis=1)
    zeros1 = jnp.zeros((NP,), jnp.float32)
    zrows = jnp.zeros((STRIPE, FC), jnp.float32)

    mesh = plsc.VectorSubcoreMesh(core_axis_name="c", subcore_axis_name="s",
                                  num_cores=2, num_subcores=16)
    degparts = _make_deg_kernel(mesh)(dstp, zeros1)

    dinv = pl.pallas_call(
        _dinv_body,
        in_specs=[pl.BlockSpec((32, NP), lambda: (0, 0))],
        out_specs=pl.BlockSpec((1, NP), lambda: (0, 0)),
        out_shape=jax.ShapeDtypeStruct((1, NP), jnp.float32),
    )(degparts)
    dinv1d = dinv.reshape(NP)

    w1d = jnp.kron(jnp.eye(28, dtype=jnp.float32), W1.astype(jnp.float32))
    mp_spec = pl.BlockSpec((NBS2, FC), lambda i: (i, 0))
    mps = pl.pallas_call(
        _mmscale_body,
        grid=(NB2,),
        in_specs=[pl.BlockSpec((NBS2, 784), lambda i: (i, 0)),
                  pl.BlockSpec((784, F), lambda i: (0, 0)),
                  pl.BlockSpec((NBS2, 1), lambda i: (i, 0))],
        out_specs=[mp_spec, mp_spec, mp_spec, mp_spec],
        out_shape=[jax.ShapeDtypeStruct((NP, FC), jnp.float32)] * 4,
    )(x, w1d, dinv.reshape(NP, 1))

    h0, h1, h2, h3, caccparts = _make_mp_kernel(mesh)(
        src3d, dst3d, mps[0], mps[1], mps[2], mps[3], dinv1d, zrows, zeros1)

    scal = pl.pallas_call(
        _w_body,
        in_specs=[pl.BlockSpec((2, NP), lambda: (0, 0)),
                  pl.BlockSpec((1, NP), lambda: (0, 0))],
        out_specs=pl.BlockSpec((2, NP), lambda: (0, 0)),
        out_shape=jax.ShapeDtypeStruct((2, NP), jnp.float32),
    )(caccparts, dinv)

    b1tile = jnp.tile(b1.astype(jnp.float32), 28)
    rs = []
    for cch, h_c in enumerate((h0, h1, h2, h3)):
        r_c = pl.pallas_call(
            _red_body,
            grid=(NB,),
            in_specs=[pl.BlockSpec((NBS, FC), lambda i: (i, 0)),
                      pl.BlockSpec((NBS, FC), lambda i: (i, 0)),
                      pl.BlockSpec((2, NP), lambda i: (0, 0)),
                      pl.BlockSpec((1, FC), lambda i: (0, 0))],
            out_specs=pl.BlockSpec((1, FC), lambda i: (0, 0)),
            out_shape=jax.ShapeDtypeStruct((1, FC), jnp.float32),
        )(h_c, mps[cch], scal,
          b1tile[cch * FC:(cch + 1) * FC].reshape(1, FC))
        rs.append(r_c)

    r28 = jnp.concatenate(rs, axis=1).reshape(28, 16)
    out = pl.pallas_call(
        _fin_body,
        in_specs=[pl.BlockSpec((28, 16), lambda: (0, 0)),
                  pl.BlockSpec((16, 32), lambda: (0, 0)),
                  pl.BlockSpec((1, 32), lambda: (0, 0)),
                  pl.BlockSpec((28, 32), lambda: (0, 0)),
                  pl.BlockSpec((1, 1), lambda: (0, 0))],
        out_specs=pl.BlockSpec((1, 1), lambda: (0, 0)),
        out_shape=jax.ShapeDtypeStruct((1, 1), jnp.float32),
    )(r28, W2.astype(jnp.float32), b2.reshape(1, 32),
      Wfc.reshape(28, 32), bfc.reshape(1, 1))
    return out


# speedup vs baseline: 1.4386x; 1.0719x over previous
import jax
import jax.numpy as jnp
from jax import lax
from jax.experimental import pallas as pl
from jax.experimental.pallas import tpu as pltpu
from jax.experimental.pallas import tpu_sc as plsc

N = 10000
NP = 10240
E0 = 100000
BATCH = 128
NBATCH = 50
NBG = NBATCH + 2
EPT = BATCH * NBATCH
EP = 16 * EPT
EPT_B = EP // 32
F = 448
FC = 112
STRIPE = NP // 16
NB = 20
NBS = 512
NB2 = 25
NBS2 = 400


def _deg_body(dst_hbm, zeros1_hbm, deg_out, dstv, accv):
    c = lax.axis_index("c")
    s = lax.axis_index("s")
    w = c * 16 + s
    pltpu.sync_copy(zeros1_hbm, accv)
    pltpu.sync_copy(dst_hbm.at[pl.ds(w * EPT_B, EPT_B)], dstv)

    def body(k, carry):
        idx = dstv[pl.ds(k * 16, 16)]
        plsc.addupdate_scatter(accv, [idx], jnp.full((16,), 1.0, jnp.float32))
        return carry

    lax.fori_loop(0, EPT_B // 16, body, 0)
    pltpu.sync_copy(accv, deg_out.at[w])


def _make_deg_kernel(mesh):
    return pl.kernel(
        _deg_body,
        out_type=jax.ShapeDtypeStruct((32, NP), jnp.float32),
        mesh=mesh,
        scratch_types=[
            pltpu.VMEM((EPT_B,), jnp.int32),
            pltpu.VMEM((NP,), jnp.float32),
        ],
        compiler_params=pltpu.CompilerParams(needs_layout_passes=False),
    )


def _run_chunk(srcv, dstv, rows0, sem0, acc_sh, zrows_hbm, stripe, mp_hbm,
               h_out):
    pltpu.sync_copy(zrows_hbm, acc_sh.at[stripe])
    plsc.subcore_barrier()

    def body(j, carry):
        pltpu.async_copy(mp_hbm.at[srcv.at[j]], rows0, sem0).wait()
        pltpu.sync_copy(rows0, acc_sh.at[dstv.at[j]], add=True)
        return carry

    lax.fori_loop(0, NBATCH, body, 0)
    plsc.subcore_barrier()
    pltpu.sync_copy(acc_sh.at[stripe], h_out.at[stripe])
    plsc.subcore_barrier()


def _mpA_body(src_hbm, dst_hbm, mpa, mpb, dinv_hbm, zrows_hbm, zeros1_hbm,
              ha_out, hb_out, cacc_out,
              srcv, dstv, rows0, valsv, sem0, acc_sh, cacc_sh):
    c = lax.axis_index("c")
    s = lax.axis_index("s")
    stripe = pl.ds(s * STRIPE, STRIPE)
    pltpu.sync_copy(src_hbm.at[s], srcv)
    pltpu.sync_copy(dst_hbm.at[s], dstv)

    def run_cacc(j_lo, row):
        pltpu.sync_copy(zeros1_hbm.at[stripe], cacc_sh.at[stripe])
        plsc.subcore_barrier()

        def body(j, carry):
            pltpu.async_copy(dinv_hbm.at[dstv.at[j]], valsv, sem0).wait()
            pltpu.sync_copy(valsv, cacc_sh.at[srcv.at[j]], add=True)
            return carry

        lax.fori_loop(j_lo, j_lo + NBATCH // 2, body, 0)
        plsc.subcore_barrier()
        pltpu.sync_copy(cacc_sh.at[stripe], cacc_out.at[row, stripe])

    @pl.when(c == 0)
    def _():
        _run_chunk(srcv, dstv, rows0, sem0, acc_sh, zrows_hbm, stripe,
                   mpa, ha_out)
        run_cacc(0, 0)

    @pl.when(c == 1)
    def _():
        _run_chunk(srcv, dstv, rows0, sem0, acc_sh, zrows_hbm, stripe,
                   mpb, hb_out)
        run_cacc(NBATCH // 2, 1)


def _mpB_body(src_hbm, dst_hbm, mpa, mpb, zrows_hbm, ha_out, hb_out,
              srcv, dstv, rows0, sem0, acc_sh):
    c = lax.axis_index("c")
    s = lax.axis_index("s")
    stripe = pl.ds(s * STRIPE, STRIPE)
    pltpu.sync_copy(src_hbm.at[s], srcv)
    pltpu.sync_copy(dst_hbm.at[s], dstv)

    @pl.when(c == 0)
    def _():
        _run_chunk(srcv, dstv, rows0, sem0, acc_sh, zrows_hbm, stripe,
                   mpa, ha_out)

    @pl.when(c == 1)
    def _():
        _run_chunk(srcv, dstv, rows0, sem0, acc_sh, zrows_hbm, stripe,
                   mpb, hb_out)


def _make_mpA_kernel(mesh):
    return pl.kernel(
        _mpA_body,
        out_type=(
            jax.ShapeDtypeStruct((NP, FC), jnp.float32),
            jax.ShapeDtypeStruct((NP, FC), jnp.float32),
            jax.ShapeDtypeStruct((2, NP), jnp.float32),
        ),
        mesh=mesh,
        scratch_types=[
            pltpu.VMEM((NBG, BATCH), jnp.int32),
            pltpu.VMEM((NBG, BATCH), jnp.int32),
            pltpu.VMEM((BATCH, FC), jnp.float32),
            pltpu.VMEM((BATCH,), jnp.float32),
            pltpu.SemaphoreType.DMA,
            pltpu.VMEM_SHARED((NP, FC), jnp.float32),
            pltpu.VMEM_SHARED((NP,), jnp.float32),
        ],
        compiler_params=pltpu.CompilerParams(needs_layout_passes=False,
                                             use_tc_tiling_on_sc=False),
    )


def _make_mpB_kernel(mesh):
    return pl.kernel(
        _mpB_body,
        out_type=(
            jax.ShapeDtypeStruct((NP, FC), jnp.float32),
            jax.ShapeDtypeStruct((NP, FC), jnp.float32),
        ),
        mesh=mesh,
        scratch_types=[
            pltpu.VMEM((NBG, BATCH), jnp.int32),
            pltpu.VMEM((NBG, BATCH), jnp.int32),
            pltpu.VMEM((BATCH, FC), jnp.float32),
            pltpu.SemaphoreType.DMA,
            pltpu.VMEM_SHARED((NP, FC), jnp.float32),
        ],
        compiler_params=pltpu.CompilerParams(needs_layout_passes=False,
                                             use_tc_tiling_on_sc=False),
    )


def _dinv_body(dp_ref, dv_ref):
    deg = jnp.sum(dp_ref[...], axis=0, keepdims=True) + 1.0
    dv_ref[...] = lax.rsqrt(deg)


def _mmpair_body(x_ref, w1p_ref, dinv_ref, mpa_ref, mpb_ref):
    mm = dinv_ref[...] * jnp.dot(x_ref[...], w1p_ref[...],
                                 preferred_element_type=jnp.float32)
    mpa_ref[...] = mm[:, :FC]
    mpb_ref[...] = mm[:, FC:]


def _w_body(cacc_ref, dinv_ref, out_ref):
    dv = dinv_ref[...]
    w = dv * jnp.sum(cacc_ref[...], axis=0, keepdims=True) + dv * dv
    lane = lax.broadcasted_iota(jnp.int32, (1, NP), 1)
    w = jnp.where(lane < N, w, 0.0)
    out_ref[...] = jnp.concatenate([dv, w], axis=0)


def _redpair_body(ha_ref, hb_ref, mpa_ref, mpb_ref, scal_ref, b1p_ref,
                  out_ref):
    i = pl.program_id(0)
    dinv = scal_ref[0, pl.ds(i * NBS, NBS)]
    w = scal_ref[1, pl.ds(i * NBS, NBS)]
    rows = i * NBS + lax.broadcasted_iota(jnp.int32, (NBS, FC), 0)

    def part(h_ref, mp_ref, b1s):
        h1 = dinv[:, None] * (h_ref[...] + mp_ref[...]) + b1s
        rl = jnp.maximum(h1, 0.0)
        rl = jnp.where(rows < N, rl, 0.0)
        return jnp.dot(w[None, :], rl, preferred_element_type=jnp.float32)

    @pl.when(i == 0)
    def _():
        out_ref[...] = jnp.zeros_like(out_ref)

    out_ref[...] += jnp.concatenate(
        [part(ha_ref, mpa_ref, b1p_ref[0:1, :]),
         part(hb_ref, mpb_ref, b1p_ref[1:2, :])], axis=1)


def _fin_body(r_ref, w2_ref, b2_ref, wfc_ref, bfc_ref, o_ref):
    h = jnp.dot(r_ref[...], w2_ref[...],
                preferred_element_type=jnp.float32) / N + b2_ref[...]
    val = jnp.sum(h * wfc_ref[...]) + bfc_ref[0, 0]
    o_ref[...] = jax.nn.sigmoid(val.reshape(1, 1) / 28.0)


def kernel(x, edge_index, W1, b1, W2, b2, Wfc, bfc):
    src = edge_index[0].astype(jnp.int32)
    dst = edge_index[1].astype(jnp.int32)
    pad = jnp.full((EP - E0,), N, jnp.int32)
    srcp = jnp.concatenate([src, pad])
    dstp = jnp.concatenate([dst, pad])
    dummy = jnp.full((16, NBG - NBATCH, BATCH), N, jnp.int32)
    src3d = jnp.concatenate([srcp.reshape(16, NBATCH, BATCH), dummy], axis=1)
    dst3d = jnp.concatenate([dstp.reshape(16, NBATCH, BATCH), dummy], axis=1)
    zeros1 = jnp.zeros((NP,), jnp.float32)
    zrows = jnp.zeros((STRIPE, FC), jnp.float32)

    mesh = plsc.VectorSubcoreMesh(core_axis_name="c", subcore_axis_name="s",
                                  num_cores=2, num_subcores=16)
    degparts = _make_deg_kernel(mesh)(dstp, zeros1)

    dinv = pl.pallas_call(
        _dinv_body,
        in_specs=[pl.BlockSpec((32, NP), lambda: (0, 0))],
        out_specs=pl.BlockSpec((1, NP), lambda: (0, 0)),
        out_shape=jax.ShapeDtypeStruct((1, NP), jnp.float32),
    )(degparts)
    dinv1d = dinv.reshape(NP)

    w1d = jnp.kron(jnp.eye(28, dtype=jnp.float32), W1.astype(jnp.float32))
    w1A = jnp.concatenate([w1d[:, 0:FC], w1d[:, 2 * FC:3 * FC]], axis=1)
    w1B = jnp.concatenate([w1d[:, FC:2 * FC], w1d[:, 3 * FC:4 * FC]], axis=1)
    mp_spec = pl.BlockSpec((NBS2, FC), lambda i: (i, 0))

    def mmpair(w1p):
        return pl.pallas_call(
            _mmpair_body,
            grid=(NB2,),
            in_specs=[pl.BlockSpec((NBS2, 784), lambda i: (i, 0)),
                      pl.BlockSpec((784, 2 * FC), lambda i: (0, 0)),
                      pl.BlockSpec((NBS2, 1), lambda i: (i, 0))],
            out_specs=[mp_spec, mp_spec],
            out_shape=[jax.ShapeDtypeStruct((NP, FC), jnp.float32)] * 2,
        )(x, w1p, dinv.reshape(NP, 1))

    mp0, mp2 = mmpair(w1A)
    h0, h2, caccparts = _make_mpA_kernel(mesh)(
        src3d, dst3d, mp0, mp2, dinv1d, zrows, zeros1)
    mp1, mp3 = mmpair(w1B)
    h1, h3 = _make_mpB_kernel(mesh)(src3d, dst3d, mp1, mp3, zrows)

    scal = pl.pallas_call(
        _w_body,
        in_specs=[pl.BlockSpec((2, NP), lambda: (0, 0)),
                  pl.BlockSpec((1, NP), lambda: (0, 0))],
        out_specs=pl.BlockSpec((2, NP), lambda: (0, 0)),
        out_shape=jax.ShapeDtypeStruct((2, NP), jnp.float32),
    )(caccparts, dinv)

    b1tile = jnp.tile(b1.astype(jnp.float32), 28)
    b1A = jnp.stack([b1tile[0:FC], b1tile[2 * FC:3 * FC]])
    b1B = jnp.stack([b1tile[FC:2 * FC], b1tile[3 * FC:4 * FC]])

    def redpair(ha, hb, mpa, mpb, b1p):
        return pl.pallas_call(
            _redpair_body,
            grid=(NB,),
            in_specs=[pl.BlockSpec((NBS, FC), lambda i: (i, 0)),
                      pl.BlockSpec((NBS, FC), lambda i: (i, 0)),
                      pl.BlockSpec((NBS, FC), lambda i: (i, 0)),
                      pl.BlockSpec((NBS, FC), lambda i: (i, 0)),
                      pl.BlockSpec((2, NP), lambda i: (0, 0)),
                      pl.BlockSpec((2, FC), lambda i: (0, 0))],
            out_specs=pl.BlockSpec((1, 2 * FC), lambda i: (0, 0)),
            out_shape=jax.ShapeDtypeStruct((1, 2 * FC), jnp.float32),
        )(ha, hb, mpa, mpb, scal, b1p)

    rA = redpair(h0, h2, mp0, mp2, b1A)
    rB = redpair(h1, h3, mp1, mp3, b1B)
    r448 = jnp.concatenate([rA[:, :FC], rB[:, :FC], rA[:, FC:], rB[:, FC:]],
                           axis=1)
    r28 = r448.reshape(28, 16)
    out = pl.pallas_call(
        _fin_body,
        in_specs=[pl.BlockSpec((28, 16), lambda: (0, 0)),
                  pl.BlockSpec((16, 32), lambda: (0, 0)),
                  pl.BlockSpec((1, 32), lambda: (0, 0)),
                  pl.BlockSpec((28, 32), lambda: (0, 0)),
                  pl.BlockSpec((1, 1), lambda: (0, 0))],
        out_specs=pl.BlockSpec((1, 1), lambda: (0, 0)),
        out_shape=jax.ShapeDtypeStruct((1, 1), jnp.float32),
    )(r28, W2.astype(jnp.float32), b2.reshape(1, 32),
      Wfc.reshape(28, 32), bfc.reshape(1, 1))
    return out


# speedup vs baseline: 1.4403x; 1.0012x over previous
import jax
import jax.numpy as jnp
from jax import lax
from jax.experimental import pallas as pl
from jax.experimental.pallas import tpu as pltpu
from jax.experimental.pallas import tpu_sc as plsc

N = 10000
NP = 10240
E0 = 100000
BATCH = 128
NBATCH = 50
NBG = NBATCH + 2
EPT = BATCH * NBATCH
EP = 16 * EPT
EPT_B = EP // 32
F = 448
FC = 112
STRIPE = NP // 16
NB = 20
NBS = 512
NB2 = 25
NBS2 = 400


def _deg_body(dst_hbm, zeros1_hbm, deg_out, dstv, accv):
    c = lax.axis_index("c")
    s = lax.axis_index("s")
    w = c * 16 + s
    pltpu.sync_copy(zeros1_hbm, accv)
    pltpu.sync_copy(dst_hbm.at[pl.ds(w * EPT_B, EPT_B)], dstv)

    def body(k, carry):
        idx = dstv[pl.ds(k * 16, 16)]
        plsc.addupdate_scatter(accv, [idx], jnp.full((16,), 1.0, jnp.float32))
        return carry

    lax.fori_loop(0, EPT_B // 16, body, 0)
    pltpu.sync_copy(accv, deg_out.at[w])


def _make_deg_kernel(mesh):
    return pl.kernel(
        _deg_body,
        out_type=jax.ShapeDtypeStruct((32, NP), jnp.float32),
        mesh=mesh,
        scratch_types=[
            pltpu.VMEM((EPT_B,), jnp.int32),
            pltpu.VMEM((NP,), jnp.float32),
        ],
        compiler_params=pltpu.CompilerParams(needs_layout_passes=False),
    )


def _run_chunk(srcv, dstv, rows0, sem0, acc_sh, zrows_hbm, stripe, mp_hbm,
               h_out):
    pltpu.sync_copy(zrows_hbm, acc_sh.at[stripe])
    plsc.subcore_barrier()

    def body(j, carry):
        pltpu.async_copy(mp_hbm.at[srcv.at[j]], rows0, sem0).wait()
        pltpu.sync_copy(rows0, acc_sh.at[dstv.at[j]], add=True)
        return carry

    lax.fori_loop(0, NBATCH, body, 0)
    plsc.subcore_barrier()
    pltpu.sync_copy(acc_sh.at[stripe], h_out.at[stripe])
    plsc.subcore_barrier()


def _mpA_body(src_hbm, dst_hbm, mpa, mpb, dinv_hbm, zrows_hbm, zeros1_hbm,
              ha_out, hb_out, cacc_out,
              srcv, dstv, rows0, valsv, sem0, acc_sh, cacc_sh):
    c = lax.axis_index("c")
    s = lax.axis_index("s")
    stripe = pl.ds(s * STRIPE, STRIPE)
    pltpu.sync_copy(src_hbm.at[s], srcv)
    pltpu.sync_copy(dst_hbm.at[s], dstv)

    def run_cacc(j_lo, row):
        pltpu.sync_copy(zeros1_hbm.at[stripe], cacc_sh.at[stripe])
        plsc.subcore_barrier()

        def body(j, carry):
            pltpu.async_copy(dinv_hbm.at[dstv.at[j]], valsv, sem0).wait()
            pltpu.sync_copy(valsv, cacc_sh.at[srcv.at[j]], add=True)
            return carry

        lax.fori_loop(j_lo, j_lo + NBATCH // 2, body, 0)
        plsc.subcore_barrier()
        pltpu.sync_copy(cacc_sh.at[stripe], cacc_out.at[row, stripe])

    @pl.when(c == 0)
    def _():
        _run_chunk(srcv, dstv, rows0, sem0, acc_sh, zrows_hbm, stripe,
                   mpa, ha_out)
        run_cacc(0, 0)

    @pl.when(c == 1)
    def _():
        _run_chunk(srcv, dstv, rows0, sem0, acc_sh, zrows_hbm, stripe,
                   mpb, hb_out)
        run_cacc(NBATCH // 2, 1)


def _mpB_body(src_hbm, dst_hbm, mpa, mpb, zrows_hbm, ha_out, hb_out,
              srcv, dstv, rows0, sem0, acc_sh):
    c = lax.axis_index("c")
    s = lax.axis_index("s")
    stripe = pl.ds(s * STRIPE, STRIPE)
    pltpu.sync_copy(src_hbm.at[s], srcv)
    pltpu.sync_copy(dst_hbm.at[s], dstv)

    @pl.when(c == 0)
    def _():
        _run_chunk(srcv, dstv, rows0, sem0, acc_sh, zrows_hbm, stripe,
                   mpa, ha_out)

    @pl.when(c == 1)
    def _():
        _run_chunk(srcv, dstv, rows0, sem0, acc_sh, zrows_hbm, stripe,
                   mpb, hb_out)


def _make_mpA_kernel(mesh):
    return pl.kernel(
        _mpA_body,
        out_type=(
            jax.ShapeDtypeStruct((NP, FC), jnp.float32),
            jax.ShapeDtypeStruct((NP, FC), jnp.float32),
            jax.ShapeDtypeStruct((2, NP), jnp.float32),
        ),
        mesh=mesh,
        scratch_types=[
            pltpu.VMEM((NBG, BATCH), jnp.int32),
            pltpu.VMEM((NBG, BATCH), jnp.int32),
            pltpu.VMEM((BATCH, FC), jnp.float32),
            pltpu.VMEM((BATCH,), jnp.float32),
            pltpu.SemaphoreType.DMA,
            pltpu.VMEM_SHARED((NP, FC), jnp.float32),
            pltpu.VMEM_SHARED((NP,), jnp.float32),
        ],
        compiler_params=pltpu.CompilerParams(needs_layout_passes=False,
                                             use_tc_tiling_on_sc=False),
    )


def _make_mpB_kernel(mesh):
    return pl.kernel(
        _mpB_body,
        out_type=(
            jax.ShapeDtypeStruct((NP, FC), jnp.float32),
            jax.ShapeDtypeStruct((NP, FC), jnp.float32),
        ),
        mesh=mesh,
        scratch_types=[
            pltpu.VMEM((NBG, BATCH), jnp.int32),
            pltpu.VMEM((NBG, BATCH), jnp.int32),
            pltpu.VMEM((BATCH, FC), jnp.float32),
            pltpu.SemaphoreType.DMA,
            pltpu.VMEM_SHARED((NP, FC), jnp.float32),
        ],
        compiler_params=pltpu.CompilerParams(needs_layout_passes=False,
                                             use_tc_tiling_on_sc=False),
    )


def _dinv_body(dp_ref, dv_ref):
    deg = jnp.sum(dp_ref[...], axis=0, keepdims=True) + 1.0
    dv_ref[...] = lax.rsqrt(deg)


def _mmpair_body(x_ref, w1p_ref, dinv_ref, mpa_ref, mpb_ref):
    mm = dinv_ref[...] * jnp.dot(x_ref[...], w1p_ref[...],
                                 preferred_element_type=jnp.float32)
    mpa_ref[...] = mm[:, :FC]
    mpb_ref[...] = mm[:, FC:]


def _redpair_body(ha_ref, hb_ref, mpa_ref, mpb_ref, cacc_ref, dinv_ref,
                  b1p_ref, out_ref):
    i = pl.program_id(0)
    dinv = dinv_ref[0, :]
    w = dinv * jnp.sum(cacc_ref[...], axis=0) + dinv * dinv
    rows = i * NBS + lax.broadcasted_iota(jnp.int32, (NBS, FC), 0)

    def part(h_ref, mp_ref, b1s):
        h1 = dinv[:, None] * (h_ref[...] + mp_ref[...]) + b1s
        rl = jnp.maximum(h1, 0.0)
        rl = jnp.where(rows < N, rl, 0.0)
        return jnp.dot(w[None, :], rl, preferred_element_type=jnp.float32)

    @pl.when(i == 0)
    def _():
        out_ref[...] = jnp.zeros_like(out_ref)

    out_ref[...] += jnp.concatenate(
        [part(ha_ref, mpa_ref, b1p_ref[0:1, :]),
         part(hb_ref, mpb_ref, b1p_ref[1:2, :])], axis=1)


def _fin_body(r_ref, w2_ref, b2_ref, wfc_ref, bfc_ref, o_ref):
    h = jnp.dot(r_ref[...], w2_ref[...],
                preferred_element_type=jnp.float32) / N + b2_ref[...]
    val = jnp.sum(h * wfc_ref[...]) + bfc_ref[0, 0]
    o_ref[...] = jax.nn.sigmoid(val.reshape(1, 1) / 28.0)


def kernel(x, edge_index, W1, b1, W2, b2, Wfc, bfc):
    src = edge_index[0].astype(jnp.int32)
    dst = edge_index[1].astype(jnp.int32)
    pad = jnp.full((EP - E0,), N, jnp.int32)
    srcp = jnp.concatenate([src, pad])
    dstp = jnp.concatenate([dst, pad])
    dummy = jnp.full((16, NBG - NBATCH, BATCH), N, jnp.int32)
    src3d = jnp.concatenate([srcp.reshape(16, NBATCH, BATCH), dummy], axis=1)
    dst3d = jnp.concatenate([dstp.reshape(16, NBATCH, BATCH), dummy], axis=1)
    zeros1 = jnp.zeros((NP,), jnp.float32)
    zrows = jnp.zeros((STRIPE, FC), jnp.float32)

    mesh = plsc.VectorSubcoreMesh(core_axis_name="c", subcore_axis_name="s",
                                  num_cores=2, num_subcores=16)
    degparts = _make_deg_kernel(mesh)(dstp, zeros1)

    dinv = pl.pallas_call(
        _dinv_body,
        in_specs=[pl.BlockSpec((32, NP), lambda: (0, 0))],
        out_specs=pl.BlockSpec((1, NP), lambda: (0, 0)),
        out_shape=jax.ShapeDtypeStruct((1, NP), jnp.float32),
    )(degparts)
    dinv1d = dinv.reshape(NP)

    w1d = jnp.kron(jnp.eye(28, dtype=jnp.float32), W1.astype(jnp.float32))
    w1A = jnp.concatenate([w1d[:, 0:FC], w1d[:, 2 * FC:3 * FC]], axis=1)
    w1B = jnp.concatenate([w1d[:, FC:2 * FC], w1d[:, 3 * FC:4 * FC]], axis=1)
    mp_spec = pl.BlockSpec((NBS2, FC), lambda i: (i, 0))

    def mmpair(w1p):
        return pl.pallas_call(
            _mmpair_body,
            grid=(NB2,),
            in_specs=[pl.BlockSpec((NBS2, 784), lambda i: (i, 0)),
                      pl.BlockSpec((784, 2 * FC), lambda i: (0, 0)),
                      pl.BlockSpec((NBS2, 1), lambda i: (i, 0))],
            out_specs=[mp_spec, mp_spec],
            out_shape=[jax.ShapeDtypeStruct((NP, FC), jnp.float32)] * 2,
        )(x, w1p, dinv.reshape(NP, 1))

    mp0, mp2 = mmpair(w1A)
    h0, h2, caccparts = _make_mpA_kernel(mesh)(
        src3d, dst3d, mp0, mp2, dinv1d, zrows, zeros1)
    mp1, mp3 = mmpair(w1B)
    h1, h3 = _make_mpB_kernel(mesh)(src3d, dst3d, mp1, mp3, zrows)

    b1tile = jnp.tile(b1.astype(jnp.float32), 28)
    b1A = jnp.stack([b1tile[0:FC], b1tile[2 * FC:3 * FC]])
    b1B = jnp.stack([b1tile[FC:2 * FC], b1tile[3 * FC:4 * FC]])

    def redpair(ha, hb, mpa, mpb, b1p):
        return pl.pallas_call(
            _redpair_body,
            grid=(NB,),
            in_specs=[pl.BlockSpec((NBS, FC), lambda i: (i, 0)),
                      pl.BlockSpec((NBS, FC), lambda i: (i, 0)),
                      pl.BlockSpec((NBS, FC), lambda i: (i, 0)),
                      pl.BlockSpec((NBS, FC), lambda i: (i, 0)),
                      pl.BlockSpec((2, NBS), lambda i: (0, i)),
                      pl.BlockSpec((1, NBS), lambda i: (0, i)),
                      pl.BlockSpec((2, FC), lambda i: (0, 0))],
            out_specs=pl.BlockSpec((1, 2 * FC), lambda i: (0, 0)),
            out_shape=jax.ShapeDtypeStruct((1, 2 * FC), jnp.float32),
        )(ha, hb, mpa, mpb, caccparts, dinv, b1p)

    rA = redpair(h0, h2, mp0, mp2, b1A)
    rB = redpair(h1, h3, mp1, mp3, b1B)
    r448 = jnp.concatenate([rA[:, :FC], rB[:, :FC], rA[:, FC:], rB[:, FC:]],
                           axis=1)
    r28 = r448.reshape(28, 16)
    out = pl.pallas_call(
        _fin_body,
        in_specs=[pl.BlockSpec((28, 16), lambda: (0, 0)),
                  pl.BlockSpec((16, 32), lambda: (0, 0)),
                  pl.BlockSpec((1, 32), lambda: (0, 0)),
                  pl.BlockSpec((28, 32), lambda: (0, 0)),
                  pl.BlockSpec((1, 1), lambda: (0, 0))],
        out_specs=pl.BlockSpec((1, 1), lambda: (0, 0)),
        out_shape=jax.ShapeDtypeStruct((1, 1), jnp.float32),
    )(r28, W2.astype(jnp.float32), b2.reshape(1, 32),
      Wfc.reshape(28, 32), bfc.reshape(1, 1))
    return out
